# baseline (speedup 1.0000x reference)
import functools

import jax
import jax.numpy as jnp
from jax import lax
from jax.experimental import pallas as pl
from jax.experimental.pallas import tpu as pltpu
from jax.experimental.pallas import tpu_sc as plsc

_L = 16
_NV = 4


@functools.lru_cache(maxsize=None)
def _make_sc_kernel(BC: int, N: int, d: int, R: int):
    NC, NS = 2, 16
    NW = NC * NS
    pairs = BC // NW
    nfull, rem = divmod(N, R)
    assert BC % NW == 0 and d % _L == 0 and pairs <= _L and R % 4 == 0
    nvec = d // _L
    mesh = plsc.VectorSubcoreMesh(core_axis_name="c", subcore_axis_name="s")

    @functools.partial(
        pl.kernel,
        out_type=jax.ShapeDtypeStruct((BC * N, d), jnp.float32),
        mesh=mesh,
        scratch_types=[
            pltpu.VMEM((2 * _L,), jnp.int32),
            pltpu.VMEM((_L,), jnp.int32),
            pltpu.VMEM((_NV, d), jnp.float32),
            pltpu.VMEM((_NV * R, d), jnp.float32),
            pltpu.SemaphoreType.DMA,
            pltpu.SemaphoreType.DMA,
            pltpu.SemaphoreType.DMA,
        ],
    )
    def k(table_hbm, im_hbm, out_hbm, im_v, sel_v, tab_v, tiles, lsem, ssem,
          ssem1):
        wid = lax.axis_index("s") * NC + lax.axis_index("c")
        base = wid * pairs
        c1 = pltpu.async_copy(im_hbm.at[pl.ds(base, _L)], im_v.at[pl.ds(0, _L)],
                              lsem)
        c2 = pltpu.async_copy(
            im_hbm.at[pl.ds(BC + base, _L)], im_v.at[pl.ds(_L, _L)], lsem)
        c3 = pltpu.async_copy(table_hbm, tab_v, lsem)
        c3.wait()

        for v in range(_NV):
            rv = [tab_v[v, pl.ds(j * _L, _L)] for j in range(nvec)]

            def body(r4, carry, v=v, rv=rv):
                for u in range(4):
                    for j in range(nvec):
                        tiles[v * R + r4 * 4 + u, pl.ds(j * _L, _L)] = rv[j]
                return carry

            lax.fori_loop(0, R // 4, body, 0)

        c1.wait()
        c2.wait()
        iv = im_v[pl.ds(0, _L)]
        mv = im_v[pl.ds(_L, _L)]
        sel_v[...] = jnp.where(mv != 0, jnp.full_like(iv, _NV - 1), iv)
        sv = sel_v[...]

        sems = (ssem, ssem1)
        inflight = []
        for p in range(pairs):
            src0 = sv[p] * R
            row0 = (base + p) * N
            for s in range(nfull):
                inflight.append(
                    pltpu.async_copy(
                        tiles.at[pl.ds(src0, R)],
                        out_hbm.at[pl.ds(row0 + s * R, R)],
                        sems[s % 2]))
            if rem:
                inflight.append(
                    pltpu.async_copy(
                        tiles.at[pl.ds(src0, rem)],
                        out_hbm.at[pl.ds(row0 + nfull * R, rem)], ssem))
        for c in inflight:
            c.wait()

    return k


def kernel(x, emb_table, indices, mask):
    B, C, N, _ = x.shape
    d = emb_table.shape[1]
    BC = B * C
    table4 = jnp.concatenate(
        [emb_table, jnp.zeros((1, d), emb_table.dtype)], axis=0)
    im = jnp.concatenate(
        [indices.reshape(BC).astype(jnp.int32),
         mask.reshape(BC).astype(jnp.int32),
         jnp.zeros((_L,), jnp.int32)])
    out = _make_sc_kernel(BC, N, d, 128)(table4, im)
    return out.reshape(B, C, N, d)


# speedup vs baseline: 1.0071x; 1.0071x over previous
import functools

import jax
import jax.numpy as jnp
from jax import lax
from jax.experimental import pallas as pl
from jax.experimental.pallas import tpu as pltpu
from jax.experimental.pallas import tpu_sc as plsc

_L = 16
_NV = 4


@functools.lru_cache(maxsize=None)
def _make_sc_kernel(BC: int, N: int, d: int, R: int):
    NC, NS = 2, 16
    NW = NC * NS
    pairs = BC // NW
    nfull, rem = divmod(N, R)
    assert BC % NW == 0 and d % _L == 0 and pairs <= _L and R % 4 == 0
    nvec = d // _L
    mesh = plsc.VectorSubcoreMesh(core_axis_name="c", subcore_axis_name="s")

    @functools.partial(
        pl.kernel,
        out_type=jax.ShapeDtypeStruct((BC * N, d), jnp.float32),
        mesh=mesh,
        scratch_types=[
            pltpu.VMEM((2 * _L,), jnp.int32),
            pltpu.VMEM((_L,), jnp.int32),
            pltpu.VMEM((_NV, d), jnp.float32),
            pltpu.VMEM((_NV * R, d), jnp.float32),
            pltpu.SemaphoreType.DMA,
            pltpu.SemaphoreType.DMA,
            pltpu.SemaphoreType.DMA,
        ],
    )
    def k(table_hbm, im_hbm, out_hbm, im_v, sel_v, tab_v, tiles, lsem, ssem,
          ssem1):
        wid = lax.axis_index("s") * NC + lax.axis_index("c")
        base = wid * pairs
        c1 = pltpu.async_copy(im_hbm.at[pl.ds(base, _L)], im_v.at[pl.ds(0, _L)],
                              lsem)
        c2 = pltpu.async_copy(
            im_hbm.at[pl.ds(BC + base, _L)], im_v.at[pl.ds(_L, _L)], lsem)
        c3 = pltpu.async_copy(table_hbm, tab_v, lsem)
        c3.wait()

        for v in range(_NV):
            rv = [tab_v[v, pl.ds(j * _L, _L)] for j in range(nvec)]

            def body(r4, carry, v=v, rv=rv):
                for u in range(4):
                    for j in range(nvec):
                        tiles[v * R + r4 * 4 + u, pl.ds(j * _L, _L)] = rv[j]
                return carry

            lax.fori_loop(0, R // 4, body, 0)

        c1.wait()
        c2.wait()
        iv = im_v[pl.ds(0, _L)]
        mv = im_v[pl.ds(_L, _L)]
        sel_v[...] = jnp.where(mv != 0, jnp.full_like(iv, _NV - 1), iv)
        sv = sel_v[...]

        sems = (ssem, ssem1)
        inflight = []
        for p in range(pairs):
            src0 = sv[p] * R
            row0 = (base + p) * N
            for s in range(nfull):
                inflight.append(
                    pltpu.async_copy(
                        tiles.at[pl.ds(src0, R)],
                        out_hbm.at[pl.ds(row0 + s * R, R)],
                        sems[0]))
            if rem:
                inflight.append(
                    pltpu.async_copy(
                        tiles.at[pl.ds(src0, rem)],
                        out_hbm.at[pl.ds(row0 + nfull * R, rem)], ssem))
        for c in inflight:
            c.wait()

    return k


def kernel(x, emb_table, indices, mask):
    B, C, N, _ = x.shape
    d = emb_table.shape[1]
    BC = B * C
    table4 = jnp.concatenate(
        [emb_table, jnp.zeros((1, d), emb_table.dtype)], axis=0)
    im = jnp.concatenate(
        [indices.reshape(BC).astype(jnp.int32),
         mask.reshape(BC).astype(jnp.int32),
         jnp.zeros((_L,), jnp.int32)])
    out = _make_sc_kernel(BC, N, d, 64)(table4, im)
    return out.reshape(B, C, N, d)


# speedup vs baseline: 1.0121x; 1.0050x over previous
import functools

import jax
import jax.numpy as jnp
from jax import lax
from jax.experimental import pallas as pl
from jax.experimental.pallas import tpu as pltpu
from jax.experimental.pallas import tpu_sc as plsc

_L = 16
_NV = 4


@functools.lru_cache(maxsize=None)
def _make_sc_kernel(BC: int, N: int, d: int, R: int):
    NC, NS = 2, 16
    NW = NC * NS
    pairs = BC // NW
    nfull, rem = divmod(N, R)
    assert BC % NW == 0 and d % _L == 0 and pairs <= _L and R % 4 == 0
    nvec = d // _L
    mesh = plsc.VectorSubcoreMesh(core_axis_name="c", subcore_axis_name="s")

    @functools.partial(
        pl.kernel,
        out_type=jax.ShapeDtypeStruct((BC * N, d), jnp.float32),
        mesh=mesh,
        scratch_types=[
            pltpu.VMEM((2 * _L,), jnp.int32),
            pltpu.VMEM((_L,), jnp.int32),
            pltpu.VMEM((_NV, d), jnp.float32),
            pltpu.VMEM((_NV * R, d), jnp.float32),
            pltpu.SemaphoreType.DMA,
            pltpu.SemaphoreType.DMA,
            pltpu.SemaphoreType.DMA,
        ],
    )
    def k(table_hbm, im_hbm, out_hbm, im_v, sel_v, tab_v, tiles, lsem, ssem,
          ssem1):
        wid = lax.axis_index("s") * NC + lax.axis_index("c")
        base = wid * pairs
        c1 = pltpu.async_copy(im_hbm.at[pl.ds(base, _L)], im_v.at[pl.ds(0, _L)],
                              lsem)
        c2 = pltpu.async_copy(
            im_hbm.at[pl.ds(BC + base, _L)], im_v.at[pl.ds(_L, _L)], lsem)
        c3 = pltpu.async_copy(table_hbm, tab_v, lsem)
        c3.wait()

        for v in range(_NV):
            rv = [tab_v[v, pl.ds(j * _L, _L)] for j in range(nvec)]

            @plsc.parallel_loop(0, R, 4)
            def body(r4, v=v, rv=rv):
                for u in range(4):
                    for j in range(nvec):
                        tiles[v * R + r4 + u, pl.ds(j * _L, _L)] = rv[j]

        c1.wait()
        c2.wait()
        iv = im_v[pl.ds(0, _L)]
        mv = im_v[pl.ds(_L, _L)]
        sel_v[...] = jnp.where(mv != 0, jnp.full_like(iv, _NV - 1), iv)
        sv = sel_v[...]

        sems = (ssem, ssem1)
        inflight = []
        for p in range(pairs):
            src0 = sv[p] * R
            row0 = (base + p) * N
            for s in range(nfull):
                inflight.append(
                    pltpu.async_copy(
                        tiles.at[pl.ds(src0, R)],
                        out_hbm.at[pl.ds(row0 + s * R, R)],
                        sems[0]))
            if rem:
                inflight.append(
                    pltpu.async_copy(
                        tiles.at[pl.ds(src0, rem)],
                        out_hbm.at[pl.ds(row0 + nfull * R, rem)], ssem))
        for c in inflight:
            c.wait()

    return k


def kernel(x, emb_table, indices, mask):
    B, C, N, _ = x.shape
    d = emb_table.shape[1]
    BC = B * C
    table4 = jnp.concatenate(
        [emb_table, jnp.zeros((1, d), emb_table.dtype)], axis=0)
    im = jnp.concatenate(
        [indices.reshape(BC).astype(jnp.int32),
         mask.reshape(BC).astype(jnp.int32),
         jnp.zeros((_L,), jnp.int32)])
    out = _make_sc_kernel(BC, N, d, 128)(table4, im)
    return out.reshape(B, C, N, d)


# speedup vs baseline: 1.0168x; 1.0046x over previous
import functools

import jax
import jax.numpy as jnp
from jax import lax
from jax.experimental import pallas as pl
from jax.experimental.pallas import tpu as pltpu
from jax.experimental.pallas import tpu_sc as plsc

_L = 16
_NV = 4


@functools.lru_cache(maxsize=None)
def _make_sc_kernel(BC: int, N: int, d: int, R: int, n_rows: int):
    NC, NS = 2, 16
    NW = NC * NS
    pairs = BC // NW
    nfull, rem = divmod(N, R)
    assert BC % NW == 0 and d % _L == 0 and pairs <= _L and R % 4 == 0
    assert n_rows == _NV - 1 and BC >= _L
    nvec = d // _L
    mesh = plsc.VectorSubcoreMesh(core_axis_name="c", subcore_axis_name="s")

    @functools.partial(
        pl.kernel,
        out_type=jax.ShapeDtypeStruct((BC * N, d), jnp.float32),
        mesh=mesh,
        scratch_types=[
            pltpu.VMEM((_L,), jnp.int32),
            pltpu.VMEM((_L,), jnp.int32),
            pltpu.VMEM((2 * _L,), jnp.int32),
            pltpu.VMEM((n_rows, d), jnp.float32),
            pltpu.VMEM((_NV * R, d), jnp.float32),
            pltpu.SemaphoreType.DMA,
            pltpu.SemaphoreType.DMA,
        ],
    )
    def k(table_hbm, idx_hbm, msk_hbm, out_hbm, idx_v, msk_v, sel_v, tab_v,
          tiles, lsem, ssem):
        wid = lax.axis_index("s") * NC + lax.axis_index("c")
        base = wid * pairs
        off = jnp.minimum(base, BC - _L)
        delta = base - off
        ct = pltpu.async_copy(table_hbm, tab_v, lsem)
        ci = pltpu.async_copy(idx_hbm.at[pl.ds(off, _L)], idx_v, lsem)
        cm = pltpu.async_copy(msk_hbm.at[pl.ds(off, _L)], msk_v, lsem)

        zv = jnp.zeros((_L,), jnp.float32)
        rows = [[zv] * nvec for _ in range(_NV)]

        def build(v, rv):
            @plsc.parallel_loop(0, R, 4)
            def body(r4, v=v, rv=rv):
                for u in range(4):
                    for j in range(nvec):
                        tiles[v * R + r4 + u, pl.ds(j * _L, _L)] = rv[j]

        build(_NV - 1, rows[_NV - 1])
        ct.wait()
        for v in range(n_rows):
            build(v, [tab_v[v, pl.ds(j * _L, _L)] for j in range(nvec)])

        ci.wait()
        cm.wait()
        iv = idx_v[...]
        mv = msk_v[...]
        sel_v[pl.ds(0, _L)] = jnp.where(
            mv != 0, jnp.full_like(iv, _NV - 1), iv) * R

        inflight = []
        for p in range(pairs):
            src0 = sel_v[pl.ds(delta + p, _L)][0]
            row0 = (base + p) * N
            for s in range(nfull):
                inflight.append(
                    pltpu.async_copy(
                        tiles.at[pl.ds(src0, R)],
                        out_hbm.at[pl.ds(row0 + s * R, R)], ssem))
            if rem:
                inflight.append(
                    pltpu.async_copy(
                        tiles.at[pl.ds(src0, rem)],
                        out_hbm.at[pl.ds(row0 + nfull * R, rem)], ssem))
        for c in inflight:
            c.wait()

    return k


def kernel(x, emb_table, indices, mask):
    B, C, N, _ = x.shape
    n_rows, d = emb_table.shape
    BC = B * C
    out = _make_sc_kernel(BC, N, d, 128, n_rows)(
        emb_table, indices.reshape(BC), mask.reshape(BC).astype(jnp.int32))
    return out.reshape(B, C, N, d)


# speedup vs baseline: 1.0320x; 1.0150x over previous
import functools

import jax
import jax.numpy as jnp
from jax import lax
from jax.experimental import pallas as pl
from jax.experimental.pallas import tpu as pltpu
from jax.experimental.pallas import tpu_sc as plsc

_L = 16
_NV = 4


@functools.lru_cache(maxsize=None)
def _make_sc_kernel(BC: int, N: int, d: int, R: int, n_rows: int):
    NC, NS = 2, 16
    NW = NC * NS
    pairs = BC // NW
    nfull, rem = divmod(N, R)
    assert BC % NW == 0 and d % _L == 0 and pairs <= _L and R % 4 == 0
    assert n_rows == _NV - 1 and BC >= _L
    nvec = d // _L
    mesh = plsc.VectorSubcoreMesh(core_axis_name="c", subcore_axis_name="s")

    @functools.partial(
        pl.kernel,
        out_type=jax.ShapeDtypeStruct((BC * N, d), jnp.float32),
        mesh=mesh,
        scratch_types=[
            pltpu.VMEM((_L,), jnp.int32),
            pltpu.VMEM((_L,), jnp.int32),
            pltpu.VMEM((2 * _L,), jnp.int32),
            pltpu.VMEM((n_rows, d), jnp.float32),
            pltpu.VMEM((_NV * R, d), jnp.float32),
            pltpu.SemaphoreType.DMA,
            pltpu.SemaphoreType.DMA,
        ],
    )
    def k(table_hbm, idx_hbm, msk_hbm, out_hbm, idx_v, msk_v, sel_v, tab_v,
          tiles, lsem, ssem):
        wid = lax.axis_index("s") * NC + lax.axis_index("c")
        base = wid * pairs
        off = jnp.minimum(base, BC - _L)
        delta = base - off
        ct = pltpu.async_copy(table_hbm, tab_v, lsem)
        ci = pltpu.async_copy(idx_hbm.at[pl.ds(off, _L)], idx_v, lsem)
        cm = pltpu.async_copy(msk_hbm.at[pl.ds(off, _L)], msk_v, lsem)

        zv = jnp.zeros((_L,), jnp.float32)
        rows = [[zv] * nvec for _ in range(_NV)]

        def build(v, rv):
            @plsc.parallel_loop(0, R, 4)
            def body(r4, v=v, rv=rv):
                for u in range(4):
                    for j in range(nvec):
                        tiles[v * R + r4 + u, pl.ds(j * _L, _L)] = rv[j]

        build(_NV - 1, rows[_NV - 1])
        ct.wait()
        for v in range(n_rows):
            build(v, [tab_v[v, pl.ds(j * _L, _L)] for j in range(nvec)])

        ci.wait()
        cm.wait()
        iv = idx_v[...]
        mv = msk_v[...]
        sel_v[pl.ds(0, _L)] = jnp.where(
            mv != 0, jnp.full_like(iv, _NV - 1), iv) * R

        assert rem == 0

        def issue_pair(p, carry):
            src0 = sel_v[pl.ds(delta + p, _L)][0]
            row0 = (base + p) * N
            for s in range(nfull):
                pltpu.async_copy(
                    tiles.at[pl.ds(src0, R)],
                    out_hbm.at[pl.ds(row0 + s * R, R)], ssem)
            return carry

        lax.fori_loop(0, pairs, issue_pair, 0)

        def drain(t, carry):
            pltpu.make_async_copy(
                tiles.at[pl.ds(0, R)], out_hbm.at[pl.ds(0, R)], ssem).wait()
            return carry

        lax.fori_loop(0, pairs * nfull, drain, 0)

    return k


def kernel(x, emb_table, indices, mask):
    B, C, N, _ = x.shape
    n_rows, d = emb_table.shape
    BC = B * C
    out = _make_sc_kernel(BC, N, d, 128, n_rows)(
        emb_table, indices.reshape(BC), mask.reshape(BC).astype(jnp.int32))
    return out.reshape(B, C, N, d)


# speedup vs baseline: 1.0376x; 1.0054x over previous
import functools

import jax
import jax.numpy as jnp
from jax import lax
from jax.experimental import pallas as pl
from jax.experimental.pallas import tpu as pltpu
from jax.experimental.pallas import tpu_sc as plsc

_L = 16
_NV = 4


@functools.lru_cache(maxsize=None)
def _make_sc_kernel(BC: int, N: int, d: int, R: int, n_rows: int):
    NC, NS = 2, 16
    NW = NC * NS
    pairs = BC // NW
    nfull, rem = divmod(N, R)
    assert BC % NW == 0 and d % _L == 0 and pairs <= _L and R % 4 == 0
    assert n_rows == _NV - 1 and BC >= _L
    nvec = d // _L
    mesh = plsc.VectorSubcoreMesh(core_axis_name="c", subcore_axis_name="s")

    @functools.partial(
        pl.kernel,
        out_type=jax.ShapeDtypeStruct((BC * N, d), jnp.float32),
        mesh=mesh,
        scratch_types=[
            pltpu.VMEM((_L,), jnp.int32),
            pltpu.VMEM((_L,), jnp.int32),
            pltpu.VMEM((2 * _L,), jnp.int32),
            pltpu.VMEM((n_rows, d), jnp.float32),
            pltpu.VMEM((_NV * R, d), jnp.float32),
            pltpu.SemaphoreType.DMA,
            pltpu.SemaphoreType.DMA,
        ],
    )
    def k(table_hbm, idx_hbm, msk_hbm, out_hbm, idx_v, msk_v, sel_v, tab_v,
          tiles, lsem, ssem):
        wid = lax.axis_index("s") * NC + lax.axis_index("c")
        base = wid * pairs
        off = jnp.minimum(base, BC - _L)
        delta = base - off
        ct = pltpu.async_copy(table_hbm, tab_v, lsem)
        ci = pltpu.async_copy(idx_hbm.at[pl.ds(off, _L)], idx_v, lsem)
        cm = pltpu.async_copy(msk_hbm.at[pl.ds(off, _L)], msk_v, lsem)

        zv = jnp.zeros((_L,), jnp.float32)
        rows = [[zv] * nvec for _ in range(_NV)]

        def build(v, rv):
            @plsc.parallel_loop(0, R, 4)
            def body(r4, v=v, rv=rv):
                for u in range(4):
                    for j in range(nvec):
                        tiles[v * R + r4 + u, pl.ds(j * _L, _L)] = rv[j]

        build(_NV - 1, rows[_NV - 1])
        ct.wait()
        for v in range(n_rows):
            build(v, [tab_v[v, pl.ds(j * _L, _L)] for j in range(nvec)])

        ci.wait()
        cm.wait()
        iv = idx_v[...]
        mv = msk_v[...]
        sel_v[pl.ds(0, _L)] = jnp.where(
            mv != 0, jnp.full_like(iv, _NV - 1), iv) * R

        assert rem == 0

        def issue_pair(p, carry):
            src0 = sel_v[pl.ds(delta + p, _L)][0]
            row0 = (base + p) * N

            def issue(s, carry2):
                pltpu.async_copy(
                    tiles.at[pl.ds(src0, R)],
                    out_hbm.at[pl.ds(row0 + s * R, R)], ssem)
                return carry2

            lax.fori_loop(0, nfull, issue, 0)
            return carry

        lax.fori_loop(0, pairs, issue_pair, 0)

        def drain(t, carry):
            pltpu.make_async_copy(
                tiles.at[pl.ds(0, R)], out_hbm.at[pl.ds(0, R)], ssem).wait()
            return carry

        lax.fori_loop(0, pairs * nfull, drain, 0)

    return k


def kernel(x, emb_table, indices, mask):
    B, C, N, _ = x.shape
    n_rows, d = emb_table.shape
    BC = B * C
    out = _make_sc_kernel(BC, N, d, 128, n_rows)(
        emb_table, indices.reshape(BC), mask.reshape(BC).astype(jnp.int32))
    return out.reshape(B, C, N, d)


# speedup vs baseline: 1.0454x; 1.0075x over previous
import functools

import jax
import jax.numpy as jnp
from jax import lax
from jax.experimental import pallas as pl
from jax.experimental.pallas import tpu as pltpu
from jax.experimental.pallas import tpu_sc as plsc

_L = 16
_NV = 4


@functools.lru_cache(maxsize=None)
def _make_sc_kernel(BC: int, N: int, d: int, R: int, n_rows: int):
    NC, NS = 2, 16
    NW = NC * NS
    pairs = BC // NW
    nfull, rem = divmod(N, R)
    assert BC % NW == 0 and d % _L == 0 and pairs <= _L and R % 4 == 0
    assert n_rows == _NV - 1 and BC >= _L
    nvec = d // _L
    mesh = plsc.VectorSubcoreMesh(core_axis_name="c", subcore_axis_name="s")

    @functools.partial(
        pl.kernel,
        out_type=jax.ShapeDtypeStruct((BC * N, d), jnp.float32),
        mesh=mesh,
        scratch_types=[
            pltpu.VMEM((_L,), jnp.int32),
            pltpu.VMEM((_L,), jnp.int32),
            pltpu.VMEM((2 * _L,), jnp.int32),
            pltpu.VMEM((n_rows, d), jnp.float32),
            pltpu.VMEM((_NV * R, d), jnp.float32),
            pltpu.SemaphoreType.DMA,
            pltpu.SemaphoreType.DMA,
        ],
    )
    def k(table_hbm, idx_hbm, msk_hbm, out_hbm, idx_v, msk_v, sel_v, tab_v,
          tiles, lsem, ssem):
        wid = lax.axis_index("s") * NC + lax.axis_index("c")
        base = wid * pairs
        off = jnp.minimum(base, BC - _L)
        delta = base - off
        ct = pltpu.async_copy(table_hbm, tab_v, lsem)
        ci = pltpu.async_copy(idx_hbm.at[pl.ds(off, _L)], idx_v, lsem)
        cm = pltpu.async_copy(msk_hbm.at[pl.ds(off, _L)], msk_v, lsem)

        zv = jnp.zeros((_L,), jnp.float32)
        rows = [[zv] * nvec for _ in range(_NV)]

        def build(v, rv):
            @plsc.parallel_loop(0, R, 4)
            def body(r4, v=v, rv=rv):
                for u in range(4):
                    for j in range(nvec):
                        tiles[v * R + r4 + u, pl.ds(j * _L, _L)] = rv[j]

        build(_NV - 1, rows[_NV - 1])
        ct.wait()
        for v in range(n_rows):
            build(v, [tab_v[v, pl.ds(j * _L, _L)] for j in range(nvec)])

        ci.wait()
        cm.wait()
        iv = idx_v[...]
        mv = msk_v[...]
        sel_v[pl.ds(0, _L)] = jnp.where(
            mv != 0, jnp.full_like(iv, _NV - 1), iv) * R

        assert rem == 0

        def issue_pair(p, carry):
            src0 = sel_v[pl.ds(delta + p, _L)][0]
            row0 = (base + p) * N

            def issue(s, carry2):
                pltpu.async_copy(
                    tiles.at[pl.ds(src0, R)],
                    out_hbm.at[pl.ds(row0 + s * R, R)], ssem)
                return carry2

            lax.fori_loop(0, nfull, issue, 0)
            return carry

        lax.fori_loop(0, pairs, issue_pair, 0)

        def drain(t, carry):
            pltpu.make_async_copy(
                tiles.at[pl.ds(0, R)], out_hbm.at[pl.ds(0, R)], ssem).wait()
            return carry

        lax.fori_loop(0, pairs * nfull, drain, 0)

    return k


def kernel(x, emb_table, indices, mask):
    B, C, N, _ = x.shape
    n_rows, d = emb_table.shape
    BC = B * C
    out = _make_sc_kernel(BC, N, d, 64, n_rows)(
        emb_table, indices.reshape(BC), mask.reshape(BC).astype(jnp.int32))
    return out.reshape(B, C, N, d)
